# baseline (speedup 1.0000x reference)
import functools

import jax
import jax.numpy as jnp
from jax import lax
from jax.experimental import pallas as pl
from jax.experimental.pallas import tpu as pltpu
from jax.experimental.pallas import tpu_sc as plsc

N, K, D_IN, D_OUT = 10000, 32, 128, 32
L = 16
NW = 32
PW = 320
CH = 1
NCHUNK = PW // CH

_INV_K = 1.0 / K


def _tc_body(f_ref, w_ref, b_ref, dm_ref, idx_ref, wo_ref, bo_ref,
             g_ref, cb_ref, wb_ref):
    acc = jnp.dot(f_ref[...], w_ref[...], preferred_element_type=jnp.float32)
    g_ref[...] = acc[:, :D_OUT]
    cb_ref[...] = jnp.concatenate(
        [acc[:, D_OUT:] + b_ref[...], dm_ref[...],
         jax.lax.bitcast_convert_type(idx_ref[...], jnp.float32),
         jnp.zeros((N, 128 - D_OUT - 2 * K), jnp.float32)], axis=1)
    row = jnp.concatenate(
        [wo_ref[...], jnp.full((1, L), bo_ref[0, 0], jnp.float32),
         jnp.zeros((1, 128 - D_OUT - L), jnp.float32)], axis=1)
    wb_ref[...] = jnp.broadcast_to(row, (8, 128))


def _tc_precompute(feature, w_cat, b_hidden, dists_max, idx, w_out, b_out):
    return pl.pallas_call(
        _tc_body,
        out_shape=(
            jax.ShapeDtypeStruct((N, D_OUT), jnp.float32),
            jax.ShapeDtypeStruct((N, 128), jnp.float32),
            jax.ShapeDtypeStruct((8, 128), jnp.float32),
        ),
    )(feature, w_cat, b_hidden.reshape(1, D_OUT), dists_max, idx,
      w_out.reshape(1, D_OUT).astype(jnp.float32),
      b_out.reshape(1, 1).astype(jnp.float32))


NBUF = 8


def _sc_body(g_hbm, cb_hbm, wb_hbm, pos_hbm, str_hbm,
             cb_v, idxf_v, pos_v, str_v, rows_v, wb_v,
             sem0, sem1, sem2, sem3, sem4, sem5, sem6, sem7, sem_in):
    cid = lax.axis_index("c")
    sid = lax.axis_index("s")
    w = sid * 2 + cid
    base = pl.multiple_of(jnp.where(w == NW - 1, N - PW, w * PW), 16)

    gsems = (sem0, sem1, sem2, sem3, sem4, sem5, sem6, sem7)
    cp_wb = pltpu.async_copy(wb_hbm, wb_v, sem_in)
    pltpu.sync_copy(cb_hbm.at[pl.ds(base, PW)], cb_v)
    cp_wb.wait()

    def issue_gather(c, b):
        off = pl.multiple_of(c * K, K)
        idxf_v[pl.ds(off, L)] = plsc.bitcast(cb_v[c, pl.ds(4 * L, L)], jnp.int32)
        idxf_v[pl.ds(off + L, L)] = plsc.bitcast(cb_v[c, pl.ds(5 * L, L)], jnp.int32)
        pltpu.async_copy(g_hbm.at[idxf_v.at[pl.ds(off, K)]], rows_v.at[b],
                         gsems[b])

    for b in range(NBUF):
        issue_gather(b, b)

    w_lo = wb_v[0, pl.ds(0, L)]
    w_hi = wb_v[0, pl.ds(L, L)]
    b_vec = wb_v[0, pl.ds(2 * L, L)]
    lane = lax.iota(jnp.int32, L)

    def pair(co, carry):
        for b in range(NBUF):
            c = co * NBUF + b
            buf = rows_v.at[b]
            pltpu.make_async_copy(
                g_hbm.at[idxf_v.at[pl.ds(pl.multiple_of(c * K, K), K)]],
                buf, gsems[b]).wait()
            for i in range(CH):
                node = c * CH + i
                s_lo = cb_v[node, pl.ds(0, L)]
                s_hi = cb_v[node, pl.ds(L, L)]
                dm_lo = cb_v[node, pl.ds(2 * L, L)]
                dm_hi = cb_v[node, pl.ds(3 * L, L)]
                acc_lo = jnp.zeros((L,), jnp.float32)
                acc_hi = jnp.zeros((L,), jnp.float32)
                p_lo = jnp.zeros((L,), jnp.float32)
                p_hi = jnp.zeros((L,), jnp.float32)
                for k in range(K):
                    g_lo = buf[i * K + k, pl.ds(0, L)]
                    g_hi = buf[i * K + k, pl.ds(L, L)]
                    a = dm_lo[k] if k < L else dm_hi[k - L]
                    h_lo = jnp.maximum(a * g_lo + s_lo, 0.0)
                    h_hi = jnp.maximum(a * g_hi + s_hi, 0.0)
                    acc_lo = acc_lo + h_lo
                    acc_hi = acc_hi + h_hi
                    pk = jnp.sum(h_lo * w_lo + h_hi * w_hi)
                    if k < L:
                        p_lo = jnp.where(lane == k, pk, p_lo)
                    else:
                        p_hi = jnp.where(lane == (k - L), pk, p_hi)
                pos_v[node, pl.ds(0, L)] = p_lo + b_vec
                pos_v[node, pl.ds(L, L)] = p_hi + b_vec
                str_v[node, pl.ds(0, L)] = acc_lo * _INV_K
                str_v[node, pl.ds(L, L)] = acc_hi * _INV_K
            @pl.when(c + NBUF < NCHUNK)
            def _():
                issue_gather(c + NBUF, b)
        return carry

    lax.fori_loop(0, NCHUNK // NBUF, pair, 0)

    pltpu.sync_copy(pos_v, pos_hbm.at[pl.ds(base, PW)])
    pltpu.sync_copy(str_v, str_hbm.at[pl.ds(base, PW)])


_sc_kernel = functools.partial(
    pl.kernel,
    out_type=(
        jax.ShapeDtypeStruct((N, K), jnp.float32),
        jax.ShapeDtypeStruct((N, D_OUT), jnp.float32),
    ),
    mesh=plsc.VectorSubcoreMesh(core_axis_name="c", subcore_axis_name="s"),
    compiler_params=pltpu.CompilerParams(
        needs_layout_passes=False, use_tc_tiling_on_sc=False),
    scratch_types=[
        pltpu.VMEM((PW, 128), jnp.float32),
        pltpu.VMEM((PW * K,), jnp.int32),
        pltpu.VMEM((PW, K), jnp.float32),
        pltpu.VMEM((PW, D_OUT), jnp.float32),
        pltpu.VMEM((NBUF, CH * K, D_OUT), jnp.float32),
        pltpu.VMEM((8, 128), jnp.float32),
        pltpu.SemaphoreType.DMA,
        pltpu.SemaphoreType.DMA,
        pltpu.SemaphoreType.DMA,
        pltpu.SemaphoreType.DMA,
        pltpu.SemaphoreType.DMA,
        pltpu.SemaphoreType.DMA,
        pltpu.SemaphoreType.DMA,
        pltpu.SemaphoreType.DMA,
        pltpu.SemaphoreType.DMA,
    ],
)(_sc_body)


def kernel(feature, dists_max, dists_argmax, W_hidden, b_hidden, W_out, b_out):
    w_cat = jnp.concatenate(
        [W_hidden[:, :D_IN].T, W_hidden[:, D_IN:].T], axis=1)
    g_tab, combo, wb8 = _tc_precompute(
        feature, w_cat, b_hidden, dists_max,
        dists_argmax.astype(jnp.int32), W_out, b_out)
    return _sc_kernel(g_tab, combo, wb8)


# speedup vs baseline: 1.0878x; 1.0878x over previous
import functools

import jax
import jax.numpy as jnp
from jax import lax
from jax.experimental import pallas as pl
from jax.experimental.pallas import tpu as pltpu
from jax.experimental.pallas import tpu_sc as plsc

N, K, D_IN, D_OUT = 10000, 32, 128, 32
L = 16
NW = 32
PW = 320
CH = 1
NCHUNK = PW // CH

_INV_K = 1.0 / K


def _tc_body(f_ref, w_ref, b_ref, dm_ref, wo_ref, bo_ref,
             g_ref, cb_ref, wb_ref):
    acc = jnp.dot(f_ref[...], w_ref[...], preferred_element_type=jnp.float32)
    g_ref[...] = acc[:, :D_OUT]
    cb_ref[...] = jnp.concatenate(
        [acc[:, D_OUT:] + b_ref[...], dm_ref[...],
         jnp.zeros((N, 128 - D_OUT - K), jnp.float32)], axis=1)
    row = jnp.concatenate(
        [wo_ref[...], jnp.full((1, L), bo_ref[0, 0], jnp.float32),
         jnp.zeros((1, 128 - D_OUT - L), jnp.float32)], axis=1)
    wb_ref[...] = jnp.broadcast_to(row, (8, 128))


def _tc_precompute(feature, w_cat, b_hidden, dists_max, w_out, b_out):
    return pl.pallas_call(
        _tc_body,
        out_shape=(
            jax.ShapeDtypeStruct((N, D_OUT), jnp.float32),
            jax.ShapeDtypeStruct((N, 128), jnp.float32),
            jax.ShapeDtypeStruct((8, 128), jnp.float32),
        ),
    )(feature, w_cat, b_hidden.reshape(1, D_OUT), dists_max,
      w_out.reshape(1, D_OUT).astype(jnp.float32),
      b_out.reshape(1, 1).astype(jnp.float32))


NBUF = 8


def _sc_body(g_hbm, cb_hbm, idx_hbm, wb_hbm, pos_hbm, str_hbm,
             idx_v, cb_v, pos_v, str_v, rows_v, wb_v,
             sem0, sem1, sem2, sem3, sem4, sem5, sem6, sem7, sem_in):
    cid = lax.axis_index("c")
    sid = lax.axis_index("s")
    w = sid * 2 + cid
    base = pl.multiple_of(jnp.where(w == NW - 1, N - PW, w * PW), 16)

    base4 = pl.multiple_of(base // CH, 4)
    gsems = (sem0, sem1, sem2, sem3, sem4, sem5, sem6, sem7)
    pltpu.sync_copy(idx_hbm.at[pl.ds(base4, NCHUNK)], idx_v)
    for b in range(NBUF):
        pltpu.async_copy(g_hbm.at[idx_v.at[b]], rows_v.at[b], gsems[b])
    cp_cb = pltpu.async_copy(cb_hbm.at[pl.ds(base, PW)], cb_v, sem_in)
    cp_wb = pltpu.async_copy(wb_hbm, wb_v, sem_in)
    cp_cb.wait()
    cp_wb.wait()

    w_lo = wb_v[0, pl.ds(0, L)]
    w_hi = wb_v[0, pl.ds(L, L)]
    b_vec = wb_v[0, pl.ds(2 * L, L)]
    lane = lax.iota(jnp.int32, L)

    def pair(co, carry):
        for b in range(NBUF):
            c = co * NBUF + b
            buf = rows_v.at[b]
            pltpu.make_async_copy(g_hbm.at[idx_v.at[c]], buf, gsems[b]).wait()
            for i in range(CH):
                node = c * CH + i
                s_lo = cb_v[node, pl.ds(0, L)]
                s_hi = cb_v[node, pl.ds(L, L)]
                dm_lo = cb_v[node, pl.ds(2 * L, L)]
                dm_hi = cb_v[node, pl.ds(3 * L, L)]
                acc_lo = jnp.zeros((L,), jnp.float32)
                acc_hi = jnp.zeros((L,), jnp.float32)
                p_lo = jnp.zeros((L,), jnp.float32)
                p_hi = jnp.zeros((L,), jnp.float32)
                for k in range(K):
                    g_lo = buf[i * K + k, pl.ds(0, L)]
                    g_hi = buf[i * K + k, pl.ds(L, L)]
                    a = dm_lo[k] if k < L else dm_hi[k - L]
                    h_lo = jnp.maximum(a * g_lo + s_lo, 0.0)
                    h_hi = jnp.maximum(a * g_hi + s_hi, 0.0)
                    acc_lo = acc_lo + h_lo
                    acc_hi = acc_hi + h_hi
                    pk = jnp.sum(h_lo * w_lo + h_hi * w_hi)
                    if k < L:
                        p_lo = jnp.where(lane == k, pk, p_lo)
                    else:
                        p_hi = jnp.where(lane == (k - L), pk, p_hi)
                pos_v[node, pl.ds(0, L)] = p_lo + b_vec
                pos_v[node, pl.ds(L, L)] = p_hi + b_vec
                str_v[node, pl.ds(0, L)] = acc_lo * _INV_K
                str_v[node, pl.ds(L, L)] = acc_hi * _INV_K
            @pl.when(c + NBUF < NCHUNK)
            def _():
                pltpu.async_copy(g_hbm.at[idx_v.at[c + NBUF]], buf, gsems[b])
        return carry

    lax.fori_loop(0, NCHUNK // NBUF, pair, 0)

    pltpu.sync_copy(pos_v, pos_hbm.at[pl.ds(base, PW)])
    pltpu.sync_copy(str_v, str_hbm.at[pl.ds(base, PW)])


_sc_kernel = functools.partial(
    pl.kernel,
    out_type=(
        jax.ShapeDtypeStruct((N, K), jnp.float32),
        jax.ShapeDtypeStruct((N, D_OUT), jnp.float32),
    ),
    mesh=plsc.VectorSubcoreMesh(core_axis_name="c", subcore_axis_name="s"),
    compiler_params=pltpu.CompilerParams(
        needs_layout_passes=False, use_tc_tiling_on_sc=False),
    scratch_types=[
        pltpu.VMEM((NCHUNK, CH * K), jnp.int32),
        pltpu.VMEM((PW, 128), jnp.float32),
        pltpu.VMEM((PW, K), jnp.float32),
        pltpu.VMEM((PW, D_OUT), jnp.float32),
        pltpu.VMEM((NBUF, CH * K, D_OUT), jnp.float32),
        pltpu.VMEM((8, 128), jnp.float32),
        pltpu.SemaphoreType.DMA,
        pltpu.SemaphoreType.DMA,
        pltpu.SemaphoreType.DMA,
        pltpu.SemaphoreType.DMA,
        pltpu.SemaphoreType.DMA,
        pltpu.SemaphoreType.DMA,
        pltpu.SemaphoreType.DMA,
        pltpu.SemaphoreType.DMA,
        pltpu.SemaphoreType.DMA,
    ],
)(_sc_body)


def kernel(feature, dists_max, dists_argmax, W_hidden, b_hidden, W_out, b_out):
    w_cat = jnp.concatenate(
        [W_hidden[:, :D_IN].T, W_hidden[:, D_IN:].T], axis=1)
    g_tab, combo, wb8 = _tc_precompute(
        feature, w_cat, b_hidden, dists_max, W_out, b_out)
    return _sc_kernel(g_tab, combo, dists_argmax.astype(jnp.int32), wb8)
